# baseline (speedup 1.0000x reference)
import functools

import jax
import jax.numpy as jnp
from jax import lax
from jax.experimental import pallas as pl
from jax.experimental.pallas import tpu as pltpu
from jax.experimental.pallas import tpu_sc as plsc

B, N, D_STATE, D_IN, M = 128, 8192, 25, 16, 128
BN = B * N
CBLK = 32768
RB = 8192

CAP = 512
NVREG_ROW = N // 16


def _logits_body(w_ref, x_ref, out_ref):
    xb = x_ref[...].astype(jnp.bfloat16)
    wb = w_ref[...].astype(jnp.bfloat16)
    out_ref[...] = lax.dot_general(
        wb, xb,
        dimension_numbers=(((1,), (0,)), ((), ())),
        preferred_element_type=jnp.float32,
    )


def _mask_body(b_ref, shift_ref, x_ref, out_ref):
    logits = x_ref[...] + b_ref[0, 0]
    p = 1.0 / (1.0 + jnp.exp(-logits))
    out_ref[...] = jnp.where(p >= 0.5, p, 0.0) + shift_ref[0, 0]


def _predicts(states2d, W, b, shift):
    xT = states2d.T
    logits = pl.pallas_call(
        _logits_body,
        grid=(BN // CBLK,),
        in_specs=[
            pl.BlockSpec((1, D_STATE), lambda i: (0, 0)),
            pl.BlockSpec((D_STATE, CBLK), lambda i: (0, i)),
        ],
        out_specs=pl.BlockSpec((1, CBLK), lambda i: (0, i)),
        out_shape=jax.ShapeDtypeStruct((1, BN), jnp.float32),
    )(W.reshape(1, D_STATE), xT)
    return pl.pallas_call(
        _mask_body,
        grid=(BN // 128 // RB,),
        in_specs=[
            pl.BlockSpec((1, 1), lambda i: (0, 0)),
            pl.BlockSpec((1, 1), lambda i: (0, 0)),
            pl.BlockSpec((RB, 128), lambda i: (i, 0)),
        ],
        out_specs=pl.BlockSpec((RB, 128), lambda i: (i, 0)),
        out_shape=jax.ShapeDtypeStruct((BN // 128, 128), jnp.float32),
    )(b.reshape(1, 1), shift.reshape(1, 1), logits.reshape(BN // 128, 128))


def _iota16():
    return lax.iota(jnp.int32, 16)


def _sc_body(p_hbm, inp_hbm, valsb_out, rows_out,
             prow, hist, valbuf, idxbuf, shbuf, keybuf, tbl,
             idxg, rows_v, valb, sem):
    nc = 2
    wid = lax.axis_index("s") * nc + lax.axis_index("c")

    def do_row(t, _):
        row = wid * 4 + t
        pltpu.sync_copy(p_hbm.at[pl.ds(row * N, N)], prow)

        def zero_hist(j, _):
            hist[pl.ds(j * 16, 16)] = jnp.zeros((16,), jnp.int32)
            return 0
        lax.fori_loop(0, 17, zero_hist, 0)

        def init_buf(j, _):
            valbuf[pl.ds(j * 16, 16)] = jnp.zeros((16,), jnp.float32)
            idxbuf[pl.ds(j * 16, 16)] = jnp.full((16,), 12288, jnp.int32)
            return 0
        lax.fori_loop(0, CAP // 16, init_buf, 0)

        def hist_body(i, _):
            v = prow[pl.ds(i * 16, 16)]
            bits = lax.bitcast_convert_type(v, jnp.int32)
            key = jnp.minimum(jnp.maximum((bits >> 15) - 32255, 0), 257)
            plsc.addupdate_scatter(hist, [key], jnp.ones((16,), jnp.int32))
            return 0
        lax.fori_loop(0, NVREG_ROW, hist_body, 0)

        def scan_body(jj, carry):
            carry_sum, tbin = carry
            j = 16 - jj
            h = hist[pl.ds(j * 16, 16)]
            binid = _iota16() + j * 16
            cs = plsc.cumsum(h)
            tot = jnp.max(cs)
            suffix = carry_sum + tot - cs + h
            cand = jnp.where(suffix >= M, binid, -1)
            return carry_sum + tot, jnp.maximum(tbin, jnp.max(cand))
        _, tbin = lax.fori_loop(0, 17, scan_body, (0, -1))
        t1 = jnp.maximum(tbin, 1)

        def cnt_body(j, carry):
            nnz = carry
            h = hist[pl.ds(j * 16, 16)]
            binid = _iota16() + j * 16
            return nnz + jnp.sum(jnp.where(binid >= 1, h, 0))
        nnz = lax.fori_loop(0, 17, cnt_body, 0)

        def compact_body(i, off):
            v = prow[pl.ds(i * 16, 16)]
            bits = lax.bitcast_convert_type(v, jnp.int32)
            key = jnp.minimum(jnp.maximum((bits >> 15) - 32255, 0), 257)
            m = (key >= t1) & (off < CAP - 16)
            plsc.store_compressed(valbuf.at[pl.ds(off, 16)], v, mask=m)
            ivec = _iota16() + i * 16
            plsc.store_compressed(idxbuf.at[pl.ds(off, 16)], ivec, mask=m)
            cnt = jnp.max(plsc.all_reduce_population_count(m))
            return off + cnt
        off = lax.fori_loop(0, NVREG_ROW, compact_body, 0)

        def zfill():
            need = M - nnz

            def zbody(i, carry):
                off2, zc = carry
                v = prow[pl.ds(i * 16, 16)]
                bits = lax.bitcast_convert_type(v, jnp.int32)
                mz = (bits == 0) & (off2 < CAP - 16)
                rank = plsc.cumsum(jnp.where(mz, 1, 0))
                m2 = mz & (zc + rank <= need)
                plsc.store_compressed(valbuf.at[pl.ds(off2, 16)], v, mask=m2)
                ivec = _iota16() + i * 16
                plsc.store_compressed(idxbuf.at[pl.ds(off2, 16)], ivec, mask=m2)
                cnt = jnp.max(plsc.all_reduce_population_count(m2))
                return off2 + cnt, zc + cnt
            return lax.fori_loop(0, NVREG_ROW, zbody, (off, 0))[0]

        off = lax.cond(nnz < M, zfill, lambda: off)
        nv = (off + 15) >> 4

        def leaf1(j, _):
            kk, vv = plsc.sort_key_val(
                valbuf[pl.ds(j * 16, 16)], idxbuf[pl.ds(j * 16, 16)],
                descending=True)
            valbuf[pl.ds(j * 16, 16)] = kk
            idxbuf[pl.ds(j * 16, 16)] = vv
            return 0
        lax.fori_loop(0, nv, leaf1, 0)

        def pass1(p, _):
            par = lax.rem(p, 2)

            def pair(jj, _):
                j = 2 * jj + par

                @pl.when(j + 1 < nv)
                def _():
                    ak = valbuf[pl.ds(j * 16, 16)]
                    av = idxbuf[pl.ds(j * 16, 16)]
                    bk = valbuf[pl.ds(j * 16 + 16, 16)]
                    bv = idxbuf[pl.ds(j * 16 + 16, 16)]
                    rbk = lax.rev(bk, (0,))
                    rbv = lax.rev(bv, (0,))
                    m = ak >= rbk
                    hk = jnp.where(m, ak, rbk)
                    hv = jnp.where(m, av, rbv)
                    lk = jnp.where(m, rbk, ak)
                    lv = jnp.where(m, rbv, av)
                    hk, hv = plsc.sort_key_val(hk, hv, descending=True)
                    lk, lv = plsc.sort_key_val(lk, lv, descending=True)
                    valbuf[pl.ds(j * 16, 16)] = hk
                    idxbuf[pl.ds(j * 16, 16)] = hv
                    valbuf[pl.ds(j * 16 + 16, 16)] = lk
                    idxbuf[pl.ds(j * 16 + 16, 16)] = lv
                return 0
            lax.fori_loop(0, (nv + 1) >> 1, pair, 0)
            return 0
        lax.fori_loop(0, nv, pass1, 0)

        shbuf[pl.ds(0, 16)] = jnp.full((16,), -1.0, jnp.float32)

        def shift_store(j, _):
            shbuf[pl.ds(j * 16 + 1, 16)] = valbuf[pl.ds(j * 16, 16)]
            return 0
        lax.fori_loop(0, nv, shift_store, 0)

        def rid_body(j, rc):
            kk = valbuf[pl.ds(j * 16, 16)]
            pv = shbuf[pl.ds(j * 16, 16)]
            neq = jnp.where(kk != pv, 1, 0)
            cs = plsc.cumsum(neq)
            rid = rc + cs
            plsc.store_scatter(tbl, [rid], kk)
            keybuf[pl.ds(j * 16, 16)] = rid * 16384 + idxbuf[pl.ds(j * 16, 16)]
            return rc + jnp.max(cs)
        lax.fori_loop(0, nv, rid_body, -1)

        def leaf2(j, _):
            kk, _vv = plsc.sort_key_val(
                keybuf[pl.ds(j * 16, 16)], keybuf[pl.ds(j * 16, 16)],
                descending=False)
            keybuf[pl.ds(j * 16, 16)] = kk
            return 0
        lax.fori_loop(0, nv, leaf2, 0)

        def pass2(p, _):
            par = lax.rem(p, 2)

            def pair(jj, _):
                j = 2 * jj + par

                @pl.when(j + 1 < nv)
                def _():
                    ak = keybuf[pl.ds(j * 16, 16)]
                    bk = keybuf[pl.ds(j * 16 + 16, 16)]
                    rbk = lax.rev(bk, (0,))
                    m = ak <= rbk
                    lk = jnp.where(m, ak, rbk)
                    hk = jnp.where(m, rbk, ak)
                    lk, _l = plsc.sort_key_val(lk, lk, descending=False)
                    hk, _h = plsc.sort_key_val(hk, hk, descending=False)
                    keybuf[pl.ds(j * 16, 16)] = lk
                    keybuf[pl.ds(j * 16 + 16, 16)] = hk
                return 0
            lax.fori_loop(0, (nv + 1) >> 1, pair, 0)
            return 0
        lax.fori_loop(0, nv, pass2, 0)

        def decode(j, _):
            key = keybuf[pl.ds(j * 16, 16)]
            idx = key & 16383
            rid = key >> 14
            val = plsc.load_gather(tbl, [rid])
            idxg[pl.ds(j * 16, 16)] = idx + row * N
            evec = _iota16() + j * 16
            for tcol in range(16):
                plsc.store_scatter(
                    valb, [evec, jnp.full((16,), tcol, jnp.int32)], val)
            return 0
        lax.fori_loop(0, M // 16, decode, 0)

        pltpu.async_copy(inp_hbm.at[idxg], rows_v, sem).wait()
        pltpu.sync_copy(rows_v, rows_out.at[pl.ds(row * M, M)])
        pltpu.sync_copy(valb, valsb_out.at[pl.ds(row * M, M)])
        return 0

    lax.fori_loop(0, 4, do_row, 0)


def _sc_topk(p_flat, inp2d):
    mesh = plsc.VectorSubcoreMesh(core_axis_name="c", subcore_axis_name="s")
    f32 = jnp.float32
    return pl.kernel(
        _sc_body,
        mesh=mesh,
        compiler_params=pltpu.CompilerParams(needs_layout_passes=False, use_tc_tiling_on_sc=False),
        out_type=(
            jax.ShapeDtypeStruct((B * M, D_IN), f32),
            jax.ShapeDtypeStruct((B * M, D_IN), f32),
        ),
        scratch_types=[
            pltpu.VMEM((N,), f32),
            pltpu.VMEM((272,), jnp.int32),
            pltpu.VMEM((CAP,), f32),
            pltpu.VMEM((CAP,), jnp.int32),
            pltpu.VMEM((CAP + 16,), f32),
            pltpu.VMEM((CAP,), jnp.int32),
            pltpu.VMEM((CAP,), f32),
            pltpu.VMEM((M,), jnp.int32),
            pltpu.VMEM((M, D_IN), f32),
            pltpu.VMEM((M, D_IN), f32),
            pltpu.SemaphoreType.DMA,
        ],
    )(p_flat, inp2d)


def _logmul_body(v_ref, r_ref, out_ref):
    la = jnp.log(jnp.minimum(jnp.maximum(v_ref[...], 1e-8), 1.0))
    out_ref[...] = r_ref[...] * la


def _logmul(valsb, rows):
    blk = 2048
    return pl.pallas_call(
        _logmul_body,
        grid=(B * M // blk,),
        in_specs=[
            pl.BlockSpec((blk, D_IN), lambda i: (i, 0)),
            pl.BlockSpec((blk, D_IN), lambda i: (i, 0)),
        ],
        out_specs=pl.BlockSpec((blk, D_IN), lambda i: (i, 0)),
        out_shape=jax.ShapeDtypeStruct((B * M, D_IN), jnp.float32),
    )(valsb, rows)


def kernel(states, inputs, W, b, k):
    shift = (jnp.asarray(k) - M).astype(jnp.float32)
    P = _predicts(states.reshape(BN, D_STATE), W, b, shift)
    valsb, rows = _sc_topk(P.reshape(BN), inputs.reshape(BN, D_IN))
    return _logmul(valsb, rows).reshape(B, M, D_IN)


# speedup vs baseline: 2.7005x; 2.7005x over previous
import functools

import jax
import jax.numpy as jnp
from jax import lax
from jax.experimental import pallas as pl
from jax.experimental.pallas import tpu as pltpu
from jax.experimental.pallas import tpu_sc as plsc

B, N, D_STATE, D_IN, M = 128, 8192, 25, 16, 128
BN = B * N
NB = 2048

CAP = 512
NVREG_ROW = N // 16


def _pred_body(w_ref, b_ref, shift_ref, x_ref, out_ref):
    x = x_ref[...]
    xb = x.astype(jnp.bfloat16)
    wb = w_ref[...].astype(jnp.bfloat16)
    outs = []
    for s in range(8):
        rhs = xb[:, s, :]
        outs.append(lax.dot_general(
            wb, rhs,
            dimension_numbers=(((1,), (0,)), ((), ())),
            preferred_element_type=jnp.float32,
        ))
    logits = jnp.concatenate(outs, axis=0) + b_ref[0, 0]
    p = 1.0 / (1.0 + jnp.exp(-logits))
    out_ref[...] = jnp.where(p >= 0.5, p, 0.0) + shift_ref[0, 0]


def _predicts(states, W, b, shift):
    sT3 = jnp.transpose(states, (2, 0, 1))
    return pl.pallas_call(
        _pred_body,
        grid=(B // 8, N // NB),
        in_specs=[
            pl.BlockSpec((1, D_STATE), lambda i, j: (0, 0)),
            pl.BlockSpec((1, 1), lambda i, j: (0, 0)),
            pl.BlockSpec((1, 1), lambda i, j: (0, 0)),
            pl.BlockSpec((D_STATE, 8, NB), lambda i, j: (0, i, j)),
        ],
        out_specs=pl.BlockSpec((8, NB), lambda i, j: (i, j)),
        out_shape=jax.ShapeDtypeStruct((B, N), jnp.float32),
    )(W.reshape(1, D_STATE), b.reshape(1, 1), shift.reshape(1, 1), sT3)


def _iota16():
    return lax.iota(jnp.int32, 16)


def _sc_body(p_hbm, inpw_hbm, valsb_out, rows_out,
             prow, hist, valbuf, idxbuf, shbuf, keybuf, tbl,
             idxw, rowsw, valb, sem):
    nc = 2
    wid = lax.axis_index("s") * nc + lax.axis_index("c")

    def do_row(t, _):
        row = wid * 4 + t
        pltpu.sync_copy(p_hbm.at[pl.ds(row * N, N)], prow)

        def zero_hist(j, _):
            hist[pl.ds(j * 16, 16)] = jnp.zeros((16,), jnp.int32)
            return 0
        lax.fori_loop(0, 17, zero_hist, 0)

        def init_buf(j, _):
            valbuf[pl.ds(j * 16, 16)] = jnp.zeros((16,), jnp.float32)
            idxbuf[pl.ds(j * 16, 16)] = jnp.full((16,), 12288, jnp.int32)
            return 0
        lax.fori_loop(0, CAP // 16, init_buf, 0)

        def hist_body(i, _):
            v = prow[pl.ds(i * 16, 16)]
            bits = lax.bitcast_convert_type(v, jnp.int32)
            key = jnp.minimum(jnp.maximum((bits >> 15) - 32255, 0), 257)
            plsc.addupdate_scatter(hist, [key], jnp.ones((16,), jnp.int32))
            return 0
        lax.fori_loop(0, NVREG_ROW, hist_body, 0)

        def scan_body(jj, carry):
            carry_sum, tbin = carry
            j = 16 - jj
            h = hist[pl.ds(j * 16, 16)]
            binid = _iota16() + j * 16
            cs = plsc.cumsum(h)
            tot = jnp.max(cs)
            suffix = carry_sum + tot - cs + h
            cand = jnp.where(suffix >= M, binid, -1)
            return carry_sum + tot, jnp.maximum(tbin, jnp.max(cand))
        _, tbin = lax.fori_loop(0, 17, scan_body, (0, -1))
        t1 = jnp.maximum(tbin, 1)

        def cnt_body(j, carry):
            h = hist[pl.ds(j * 16, 16)]
            binid = _iota16() + j * 16
            return carry + jnp.sum(jnp.where(binid >= 1, h, 0))
        nnz = lax.fori_loop(0, 17, cnt_body, 0)

        def compact_body(i, off):
            v = prow[pl.ds(i * 16, 16)]
            bits = lax.bitcast_convert_type(v, jnp.int32)
            key = jnp.minimum(jnp.maximum((bits >> 15) - 32255, 0), 257)
            m = (key >= t1) & (off < CAP - 16)
            plsc.store_compressed(valbuf.at[pl.ds(off, 16)], v, mask=m)
            ivec = _iota16() + i * 16
            plsc.store_compressed(idxbuf.at[pl.ds(off, 16)], ivec, mask=m)
            cnt = jnp.max(plsc.all_reduce_population_count(m))
            return off + cnt
        off = lax.fori_loop(0, NVREG_ROW, compact_body, 0)

        def zfill():
            need = M - nnz

            def zbody(i, carry):
                off2, zc = carry
                v = prow[pl.ds(i * 16, 16)]
                bits = lax.bitcast_convert_type(v, jnp.int32)
                mz = (bits == 0) & (off2 < CAP - 16)
                rank = plsc.cumsum(jnp.where(mz, 1, 0))
                m2 = mz & (zc + rank <= need)
                plsc.store_compressed(valbuf.at[pl.ds(off2, 16)], v, mask=m2)
                ivec = _iota16() + i * 16
                plsc.store_compressed(idxbuf.at[pl.ds(off2, 16)], ivec, mask=m2)
                cnt = jnp.max(plsc.all_reduce_population_count(m2))
                return off2 + cnt, zc + cnt
            return lax.fori_loop(0, NVREG_ROW, zbody, (off, 0))[0]

        off2 = lax.cond(nnz < M, zfill, lambda: off)
        nv = (off2 + 15) >> 4

        def leaf1(j, _):
            kk, vv = plsc.sort_key_val(
                valbuf[pl.ds(j * 16, 16)], idxbuf[pl.ds(j * 16, 16)],
                descending=True)
            valbuf[pl.ds(j * 16, 16)] = kk
            idxbuf[pl.ds(j * 16, 16)] = vv
            return 0
        lax.fori_loop(0, nv, leaf1, 0)

        def pass1(p, _):
            par = lax.rem(p, 2)

            def pair(jj, _):
                j = 2 * jj + par

                @pl.when(j + 1 < nv)
                def _():
                    ak = valbuf[pl.ds(j * 16, 16)]
                    av = idxbuf[pl.ds(j * 16, 16)]
                    bk = valbuf[pl.ds(j * 16 + 16, 16)]
                    bv = idxbuf[pl.ds(j * 16 + 16, 16)]
                    rbk = lax.rev(bk, (0,))
                    rbv = lax.rev(bv, (0,))
                    m = ak >= rbk
                    hk = jnp.where(m, ak, rbk)
                    hv = jnp.where(m, av, rbv)
                    lk = jnp.where(m, rbk, ak)
                    lv = jnp.where(m, rbv, av)
                    hk, hv = plsc.sort_key_val(hk, hv, descending=True)
                    lk, lv = plsc.sort_key_val(lk, lv, descending=True)
                    valbuf[pl.ds(j * 16, 16)] = hk
                    idxbuf[pl.ds(j * 16, 16)] = hv
                    valbuf[pl.ds(j * 16 + 16, 16)] = lk
                    idxbuf[pl.ds(j * 16 + 16, 16)] = lv
                return 0
            lax.fori_loop(0, (nv + 1) >> 1, pair, 0)
            return 0
        lax.fori_loop(0, nv, pass1, 0)

        shbuf[pl.ds(0, 16)] = jnp.full((16,), -1.0, jnp.float32)

        def shift_store(j, _):
            shbuf[pl.ds(j * 16 + 1, 16)] = valbuf[pl.ds(j * 16, 16)]
            return 0
        lax.fori_loop(0, nv, shift_store, 0)

        def rid_body(j, rc):
            kk = valbuf[pl.ds(j * 16, 16)]
            pv = shbuf[pl.ds(j * 16, 16)]
            neq = jnp.where(kk != pv, 1, 0)
            cs = plsc.cumsum(neq)
            rid = rc + cs
            plsc.store_scatter(tbl, [rid], kk)
            keybuf[pl.ds(j * 16, 16)] = rid * 16384 + idxbuf[pl.ds(j * 16, 16)]
            return rc + jnp.max(cs)
        lax.fori_loop(0, nv, rid_body, -1)

        def leaf2(j, _):
            kk, _vv = plsc.sort_key_val(
                keybuf[pl.ds(j * 16, 16)], keybuf[pl.ds(j * 16, 16)],
                descending=False)
            keybuf[pl.ds(j * 16, 16)] = kk
            return 0
        lax.fori_loop(0, nv, leaf2, 0)

        def pass2(p, _):
            par = lax.rem(p, 2)

            def pair(jj, _):
                j = 2 * jj + par

                @pl.when(j + 1 < nv)
                def _():
                    ak = keybuf[pl.ds(j * 16, 16)]
                    bk = keybuf[pl.ds(j * 16 + 16, 16)]
                    rbk = lax.rev(bk, (0,))
                    m = ak <= rbk
                    lk = jnp.where(m, ak, rbk)
                    hk = jnp.where(m, rbk, ak)
                    lk, _l = plsc.sort_key_val(lk, lk, descending=False)
                    hk, _h = plsc.sort_key_val(hk, hk, descending=False)
                    keybuf[pl.ds(j * 16, 16)] = lk
                    keybuf[pl.ds(j * 16 + 16, 16)] = hk
                return 0
            lax.fori_loop(0, (nv + 1) >> 1, pair, 0)
            return 0
        lax.fori_loop(0, nv, pass2, 0)

        def decode(j, _):
            key = keybuf[pl.ds(j * 16, 16)]
            idx = key & 16383
            rid = key >> 14
            val = plsc.load_gather(tbl, [rid])
            evec = _iota16() + j * 16
            for tcol in range(D_IN):
                word = row * (D_IN * N) + tcol * N + idx
                slot = evec * D_IN + tcol
                plsc.store_scatter(
                    idxw, [slot >> 7, slot & 127], word)
                plsc.store_scatter(
                    valb, [evec, jnp.full((16,), tcol, jnp.int32)], val)
            return 0
        lax.fori_loop(0, M // 16, decode, 0)

        def fire(g, _):
            pltpu.async_copy(inpw_hbm.at[idxw.at[g]],
                             rowsw.at[pl.ds(g * 128, 128)], sem)
            return 0
        lax.fori_loop(0, 16, fire, 0)

        def drain(g, _):
            pltpu.make_async_copy(inpw_hbm.at[idxw.at[g]],
                                  rowsw.at[pl.ds(g * 128, 128)], sem).wait()
            return 0
        lax.fori_loop(0, 16, drain, 0)

        pltpu.sync_copy(rowsw, rows_out.at[pl.ds(row * M * D_IN, M * D_IN)])
        pltpu.sync_copy(valb, valsb_out.at[pl.ds(row * M, M)])
        return 0

    lax.fori_loop(0, 4, do_row, 0)


def _sc_topk(p_flat, inpw):
    mesh = plsc.VectorSubcoreMesh(core_axis_name="c", subcore_axis_name="s")
    f32 = jnp.float32
    return pl.kernel(
        _sc_body,
        mesh=mesh,
        compiler_params=pltpu.CompilerParams(
            needs_layout_passes=False, use_tc_tiling_on_sc=False),
        out_type=(
            jax.ShapeDtypeStruct((B * M, D_IN), f32),
            jax.ShapeDtypeStruct((B * M * D_IN,), f32),
        ),
        scratch_types=[
            pltpu.VMEM((N,), f32),
            pltpu.VMEM((272,), jnp.int32),
            pltpu.VMEM((CAP,), f32),
            pltpu.VMEM((CAP,), jnp.int32),
            pltpu.VMEM((CAP + 16,), f32),
            pltpu.VMEM((CAP,), jnp.int32),
            pltpu.VMEM((CAP,), f32),
            pltpu.VMEM((M * D_IN // 128, 128), jnp.int32),
            pltpu.VMEM((M * D_IN,), f32),
            pltpu.VMEM((M, D_IN), f32),
            pltpu.SemaphoreType.DMA,
        ],
    )(p_flat, inpw)


def _logmul_body(v_ref, r_ref, out_ref):
    la = jnp.log(jnp.minimum(jnp.maximum(v_ref[...], 1e-8), 1.0))
    out_ref[...] = r_ref[...] * la


def _logmul(valsb, rows):
    blk = 2048
    return pl.pallas_call(
        _logmul_body,
        grid=(B * M // blk,),
        in_specs=[
            pl.BlockSpec((blk, D_IN), lambda i: (i, 0)),
            pl.BlockSpec((blk, D_IN), lambda i: (i, 0)),
        ],
        out_specs=pl.BlockSpec((blk, D_IN), lambda i: (i, 0)),
        out_shape=jax.ShapeDtypeStruct((B * M, D_IN), jnp.float32),
    )(valsb, rows)


def kernel(states, inputs, W, b, k):
    shift = (jnp.asarray(k) - M).astype(jnp.float32)
    P = _predicts(states, W, b, shift)
    inpw = jnp.transpose(inputs, (0, 2, 1)).reshape(B * D_IN * N)
    valsb, rows = _sc_topk(P.reshape(BN), inpw)
    return _logmul(valsb, rows.reshape(B * M, D_IN)).reshape(B, M, D_IN)


# speedup vs baseline: 2.8533x; 1.0566x over previous
import functools

import jax
import jax.numpy as jnp
from jax import lax
from jax.experimental import pallas as pl
from jax.experimental.pallas import tpu as pltpu
from jax.experimental.pallas import tpu_sc as plsc

B, N, D_STATE, D_IN, M = 128, 8192, 25, 16, 128
BN = B * N
NB = 2048

CAP = 512
NVREG_ROW = N // 16


def _pred_body(w_ref, b_ref, shift_ref, x_ref, out_ref):
    x = x_ref[...]
    xb = x.astype(jnp.bfloat16)
    wb = w_ref[...].astype(jnp.bfloat16)
    outs = []
    for s in range(8):
        rhs = xb[:, s, :]
        outs.append(lax.dot_general(
            wb, rhs,
            dimension_numbers=(((1,), (0,)), ((), ())),
            preferred_element_type=jnp.float32,
        ))
    logits = jnp.concatenate(outs, axis=0) + b_ref[0, 0]
    p = 1.0 / (1.0 + jnp.exp(-logits))
    out_ref[...] = jnp.where(p >= 0.5, p, 0.0) + shift_ref[0, 0]


def _predicts(states, W, b, shift):
    sT3 = jnp.transpose(states, (2, 0, 1))
    return pl.pallas_call(
        _pred_body,
        grid=(B // 8, N // NB),
        in_specs=[
            pl.BlockSpec((1, D_STATE), lambda i, j: (0, 0)),
            pl.BlockSpec((1, 1), lambda i, j: (0, 0)),
            pl.BlockSpec((1, 1), lambda i, j: (0, 0)),
            pl.BlockSpec((D_STATE, 8, NB), lambda i, j: (0, i, j)),
        ],
        out_specs=pl.BlockSpec((8, NB), lambda i, j: (i, j)),
        out_shape=jax.ShapeDtypeStruct((B, N), jnp.float32),
    )(W.reshape(1, D_STATE), b.reshape(1, 1), shift.reshape(1, 1), sT3)


def _iota16():
    return lax.iota(jnp.int32, 16)


def _sc_body(p_hbm, inpw_hbm, valsb_out, rows_out,
             prow, hist, valbuf, idxbuf, shbuf, keybuf, tbl,
             idxw, rowsw, valb, sem):
    nc = 2
    wid = lax.axis_index("s") * nc + lax.axis_index("c")

    def do_row(t, _):
        row = wid * 4 + t
        pltpu.sync_copy(p_hbm.at[pl.ds(row * N, N)], prow)

        def zero_hist(j, _):
            hist[pl.ds(j * 16, 16)] = jnp.zeros((16,), jnp.int32)
            return 0
        lax.fori_loop(0, 17, zero_hist, 0)

        def init_buf(j, _):
            valbuf[pl.ds(j * 16, 16)] = jnp.zeros((16,), jnp.float32)
            idxbuf[pl.ds(j * 16, 16)] = jnp.full((16,), 12288, jnp.int32)
            return 0
        lax.fori_loop(0, CAP // 16, init_buf, 0)

        @plsc.parallel_loop(0, NVREG_ROW, unroll=8)
        def hist_body(i):
            v = prow[pl.ds(i * 16, 16)]
            bits = lax.bitcast_convert_type(v, jnp.int32)
            key = jnp.maximum((bits >> 15) - 32255, 0)
            plsc.addupdate_scatter(hist, [key], jnp.ones((16,), jnp.int32))

        def scan_body(jj, carry):
            carry_sum, tbin = carry
            j = 16 - jj
            h = hist[pl.ds(j * 16, 16)]
            binid = _iota16() + j * 16
            cs = plsc.cumsum(h)
            tot = jnp.max(cs)
            suffix = carry_sum + tot - cs + h
            cand = jnp.where(suffix >= M, binid, -1)
            return carry_sum + tot, jnp.maximum(tbin, jnp.max(cand))
        _, tbin = lax.fori_loop(0, 17, scan_body, (0, -1))
        t1 = jnp.maximum(tbin, 1)

        def cnt_body(j, carry):
            h = hist[pl.ds(j * 16, 16)]
            binid = _iota16() + j * 16
            return carry + jnp.sum(jnp.where(binid >= 1, h, 0))
        nnz = lax.fori_loop(0, 17, cnt_body, 0)

        def compact_body(i, off):
            v = prow[pl.ds(i * 16, 16)]
            bits = lax.bitcast_convert_type(v, jnp.int32)
            key = jnp.minimum(jnp.maximum((bits >> 15) - 32255, 0), 257)
            m = (key >= t1) & (off < CAP - 16)
            plsc.store_compressed(valbuf.at[pl.ds(off, 16)], v, mask=m)
            ivec = _iota16() + i * 16
            plsc.store_compressed(idxbuf.at[pl.ds(off, 16)], ivec, mask=m)
            cnt = jnp.max(plsc.all_reduce_population_count(m))
            return off + cnt
        off = lax.fori_loop(0, NVREG_ROW, compact_body, 0, unroll=4)

        def zfill():
            need = M - nnz

            def zbody(i, carry):
                off2, zc = carry
                v = prow[pl.ds(i * 16, 16)]
                bits = lax.bitcast_convert_type(v, jnp.int32)
                mz = (bits == 0) & (off2 < CAP - 16)
                rank = plsc.cumsum(jnp.where(mz, 1, 0))
                m2 = mz & (zc + rank <= need)
                plsc.store_compressed(valbuf.at[pl.ds(off2, 16)], v, mask=m2)
                ivec = _iota16() + i * 16
                plsc.store_compressed(idxbuf.at[pl.ds(off2, 16)], ivec, mask=m2)
                cnt = jnp.max(plsc.all_reduce_population_count(m2))
                return off2 + cnt, zc + cnt
            return lax.fori_loop(0, NVREG_ROW, zbody, (off, 0))[0]

        off2 = lax.cond(nnz < M, zfill, lambda: off)
        nv = (off2 + 15) >> 4

        def leaf1(j, _):
            kk, vv = plsc.sort_key_val(
                valbuf[pl.ds(j * 16, 16)], idxbuf[pl.ds(j * 16, 16)],
                descending=True)
            valbuf[pl.ds(j * 16, 16)] = kk
            idxbuf[pl.ds(j * 16, 16)] = vv
            return 0
        lax.fori_loop(0, nv, leaf1, 0)

        def pass1(p, _):
            par = lax.rem(p, 2)

            def pair(jj, _):
                j = 2 * jj + par

                @pl.when(j + 1 < nv)
                def _():
                    ak = valbuf[pl.ds(j * 16, 16)]
                    av = idxbuf[pl.ds(j * 16, 16)]
                    bk = valbuf[pl.ds(j * 16 + 16, 16)]
                    bv = idxbuf[pl.ds(j * 16 + 16, 16)]
                    rbk = lax.rev(bk, (0,))
                    rbv = lax.rev(bv, (0,))
                    m = ak >= rbk
                    hk = jnp.where(m, ak, rbk)
                    hv = jnp.where(m, av, rbv)
                    lk = jnp.where(m, rbk, ak)
                    lv = jnp.where(m, rbv, av)
                    hk, hv = plsc.sort_key_val(hk, hv, descending=True)
                    lk, lv = plsc.sort_key_val(lk, lv, descending=True)
                    valbuf[pl.ds(j * 16, 16)] = hk
                    idxbuf[pl.ds(j * 16, 16)] = hv
                    valbuf[pl.ds(j * 16 + 16, 16)] = lk
                    idxbuf[pl.ds(j * 16 + 16, 16)] = lv
                return 0
            lax.fori_loop(0, (nv + 1) >> 1, pair, 0)
            return 0
        lax.fori_loop(0, nv, pass1, 0)

        shbuf[pl.ds(0, 16)] = jnp.full((16,), -1.0, jnp.float32)

        def shift_store(j, _):
            shbuf[pl.ds(j * 16 + 1, 16)] = valbuf[pl.ds(j * 16, 16)]
            return 0
        lax.fori_loop(0, nv, shift_store, 0)

        def rid_body(j, rc):
            kk = valbuf[pl.ds(j * 16, 16)]
            pv = shbuf[pl.ds(j * 16, 16)]
            neq = jnp.where(kk != pv, 1, 0)
            cs = plsc.cumsum(neq)
            rid = rc + cs
            plsc.store_scatter(tbl, [rid], kk)
            keybuf[pl.ds(j * 16, 16)] = rid * 16384 + idxbuf[pl.ds(j * 16, 16)]
            return rc + jnp.max(cs)
        lax.fori_loop(0, nv, rid_body, -1)

        def leaf2(j, _):
            kk, _vv = plsc.sort_key_val(
                keybuf[pl.ds(j * 16, 16)], keybuf[pl.ds(j * 16, 16)],
                descending=False)
            keybuf[pl.ds(j * 16, 16)] = kk
            return 0
        lax.fori_loop(0, nv, leaf2, 0)

        def pass2(p, _):
            par = lax.rem(p, 2)

            def pair(jj, _):
                j = 2 * jj + par

                @pl.when(j + 1 < nv)
                def _():
                    ak = keybuf[pl.ds(j * 16, 16)]
                    bk = keybuf[pl.ds(j * 16 + 16, 16)]
                    rbk = lax.rev(bk, (0,))
                    m = ak <= rbk
                    lk = jnp.where(m, ak, rbk)
                    hk = jnp.where(m, rbk, ak)
                    lk, _l = plsc.sort_key_val(lk, lk, descending=False)
                    hk, _h = plsc.sort_key_val(hk, hk, descending=False)
                    keybuf[pl.ds(j * 16, 16)] = lk
                    keybuf[pl.ds(j * 16 + 16, 16)] = hk
                return 0
            lax.fori_loop(0, (nv + 1) >> 1, pair, 0)
            return 0
        lax.fori_loop(0, nv, pass2, 0)

        def decode(j, _):
            key = keybuf[pl.ds(j * 16, 16)]
            idx = key & 16383
            rid = key >> 14
            val = plsc.load_gather(tbl, [rid])
            evec = _iota16() + j * 16
            for tcol in range(D_IN):
                word = row * (D_IN * N) + tcol * N + idx
                slot = evec * D_IN + tcol
                plsc.store_scatter(
                    idxw, [slot >> 7, slot & 127], word)
                plsc.store_scatter(
                    valb, [evec, jnp.full((16,), tcol, jnp.int32)], val)
            return 0
        lax.fori_loop(0, M // 16, decode, 0)

        def fire(g, _):
            pltpu.async_copy(inpw_hbm.at[idxw.at[g]],
                             rowsw.at[pl.ds(g * 128, 128)], sem)
            return 0
        lax.fori_loop(0, 16, fire, 0)

        def drain(g, _):
            pltpu.make_async_copy(inpw_hbm.at[idxw.at[g]],
                                  rowsw.at[pl.ds(g * 128, 128)], sem).wait()
            return 0
        lax.fori_loop(0, 16, drain, 0)

        pltpu.sync_copy(rowsw, rows_out.at[pl.ds(row * M * D_IN, M * D_IN)])
        pltpu.sync_copy(valb, valsb_out.at[pl.ds(row * M, M)])
        return 0

    lax.fori_loop(0, 4, do_row, 0)


def _sc_topk(p_flat, inpw):
    mesh = plsc.VectorSubcoreMesh(core_axis_name="c", subcore_axis_name="s")
    f32 = jnp.float32
    return pl.kernel(
        _sc_body,
        mesh=mesh,
        compiler_params=pltpu.CompilerParams(
            needs_layout_passes=False, use_tc_tiling_on_sc=False),
        out_type=(
            jax.ShapeDtypeStruct((B * M, D_IN), f32),
            jax.ShapeDtypeStruct((B * M * D_IN,), f32),
        ),
        scratch_types=[
            pltpu.VMEM((N,), f32),
            pltpu.VMEM((272,), jnp.int32),
            pltpu.VMEM((CAP,), f32),
            pltpu.VMEM((CAP,), jnp.int32),
            pltpu.VMEM((CAP + 16,), f32),
            pltpu.VMEM((CAP,), jnp.int32),
            pltpu.VMEM((CAP,), f32),
            pltpu.VMEM((M * D_IN // 128, 128), jnp.int32),
            pltpu.VMEM((M * D_IN,), f32),
            pltpu.VMEM((M, D_IN), f32),
            pltpu.SemaphoreType.DMA,
        ],
    )(p_flat, inpw)


def _logmul_body(v_ref, r_ref, out_ref):
    la = jnp.log(jnp.minimum(jnp.maximum(v_ref[...], 1e-8), 1.0))
    out_ref[...] = r_ref[...] * la


def _logmul(valsb, rows):
    blk = 2048
    return pl.pallas_call(
        _logmul_body,
        grid=(B * M // blk,),
        in_specs=[
            pl.BlockSpec((blk, D_IN), lambda i: (i, 0)),
            pl.BlockSpec((blk, D_IN), lambda i: (i, 0)),
        ],
        out_specs=pl.BlockSpec((blk, D_IN), lambda i: (i, 0)),
        out_shape=jax.ShapeDtypeStruct((B * M, D_IN), jnp.float32),
    )(valsb, rows)


def kernel(states, inputs, W, b, k):
    shift = (jnp.asarray(k) - M).astype(jnp.float32)
    P = _predicts(states, W, b, shift)
    inpw = jnp.transpose(inputs, (0, 2, 1)).reshape(B * D_IN * N)
    valsb, rows = _sc_topk(P.reshape(BN), inpw)
    return _logmul(valsb, rows.reshape(B * M, D_IN)).reshape(B, M, D_IN)


# speedup vs baseline: 3.2270x; 1.1310x over previous
import functools

import jax
import jax.numpy as jnp
from jax import lax
from jax.experimental import pallas as pl
from jax.experimental.pallas import tpu as pltpu
from jax.experimental.pallas import tpu_sc as plsc

B, N, D_STATE, D_IN, M = 128, 8192, 25, 16, 128
BN = B * N
NB = 2048

CAP = 512
NVREG_ROW = N // 16


def _pred_body(w_ref, b_ref, shift_ref, x_ref, out_ref):
    x = x_ref[...]
    xb = x.astype(jnp.bfloat16)
    wb = w_ref[...].astype(jnp.bfloat16)
    outs = []
    for s in range(8):
        rhs = xb[:, s, :]
        outs.append(lax.dot_general(
            wb, rhs,
            dimension_numbers=(((1,), (0,)), ((), ())),
            preferred_element_type=jnp.float32,
        ))
    logits = jnp.concatenate(outs, axis=0) + b_ref[0, 0]
    p = 1.0 / (1.0 + jnp.exp(-logits))
    out_ref[...] = jnp.where(p >= 0.5, p, 0.0) + shift_ref[0, 0]


def _predicts(sT3, W, b, shift, half):
    hb = half * (B // 2 // 8)
    return pl.pallas_call(
        _pred_body,
        grid=(B // 2 // 8, N // NB),
        in_specs=[
            pl.BlockSpec((1, D_STATE), lambda i, j: (0, 0)),
            pl.BlockSpec((1, 1), lambda i, j: (0, 0)),
            pl.BlockSpec((1, 1), lambda i, j: (0, 0)),
            pl.BlockSpec((D_STATE, 8, NB), lambda i, j, hb=hb: (0, hb + i, j)),
        ],
        out_specs=pl.BlockSpec((8, NB), lambda i, j: (i, j)),
        out_shape=jax.ShapeDtypeStruct((B // 2, N), jnp.float32),
    )(W.reshape(1, D_STATE), b.reshape(1, 1), shift.reshape(1, 1), sT3)


def _iota16():
    return lax.iota(jnp.int32, 16)


def _sc_body(half, p_hbm, inpw_hbm, valsb_out, rows_out,
             prow, hist, valbuf, idxbuf, shbuf, keybuf, tbl,
             idxw, rowsw, valb, sem):
    nc = 2
    wid = lax.axis_index("s") * nc + lax.axis_index("c")

    def do_row(t, _):
        row = wid * 2 + t
        pltpu.sync_copy(p_hbm.at[pl.ds(row * N, N)], prow)

        def zero_hist(j, _):
            hist[pl.ds(j * 16, 16)] = jnp.zeros((16,), jnp.int32)
            return 0
        lax.fori_loop(0, 17, zero_hist, 0)

        def init_buf(j, _):
            valbuf[pl.ds(j * 16, 16)] = jnp.zeros((16,), jnp.float32)
            idxbuf[pl.ds(j * 16, 16)] = jnp.full((16,), 12288, jnp.int32)
            return 0
        lax.fori_loop(0, CAP // 16, init_buf, 0)

        @plsc.parallel_loop(0, NVREG_ROW, unroll=8)
        def hist_body(i):
            v = prow[pl.ds(i * 16, 16)]
            bits = lax.bitcast_convert_type(v, jnp.int32)
            key = jnp.maximum((bits >> 15) - 32255, 0)
            plsc.addupdate_scatter(hist, [key], jnp.ones((16,), jnp.int32))

        def scan_body(jj, carry):
            carry_sum, tbin = carry
            j = 16 - jj
            h = hist[pl.ds(j * 16, 16)]
            binid = _iota16() + j * 16
            cs = plsc.cumsum(h)
            tot = jnp.max(cs)
            suffix = carry_sum + tot - cs + h
            cand = jnp.where(suffix >= M, binid, -1)
            return carry_sum + tot, jnp.maximum(tbin, jnp.max(cand))
        _, tbin = lax.fori_loop(0, 17, scan_body, (0, -1))
        t1 = jnp.maximum(tbin, 1)

        def cnt_body(j, carry):
            h = hist[pl.ds(j * 16, 16)]
            binid = _iota16() + j * 16
            return carry + jnp.sum(jnp.where(binid >= 1, h, 0))
        nnz = lax.fori_loop(0, 17, cnt_body, 0)

        def compact_body(i, off):
            v = prow[pl.ds(i * 16, 16)]
            bits = lax.bitcast_convert_type(v, jnp.int32)
            key = jnp.minimum(jnp.maximum((bits >> 15) - 32255, 0), 257)
            m = (key >= t1) & (off < CAP - 16)
            plsc.store_compressed(valbuf.at[pl.ds(off, 16)], v, mask=m)
            ivec = _iota16() + i * 16
            plsc.store_compressed(idxbuf.at[pl.ds(off, 16)], ivec, mask=m)
            cnt = jnp.max(plsc.all_reduce_population_count(m))
            return off + cnt
        off = lax.fori_loop(0, NVREG_ROW, compact_body, 0, unroll=4)

        def zfill():
            need = M - nnz

            def zbody(i, carry):
                off2, zc = carry
                v = prow[pl.ds(i * 16, 16)]
                bits = lax.bitcast_convert_type(v, jnp.int32)
                mz = (bits == 0) & (off2 < CAP - 16)
                rank = plsc.cumsum(jnp.where(mz, 1, 0))
                m2 = mz & (zc + rank <= need)
                plsc.store_compressed(valbuf.at[pl.ds(off2, 16)], v, mask=m2)
                ivec = _iota16() + i * 16
                plsc.store_compressed(idxbuf.at[pl.ds(off2, 16)], ivec, mask=m2)
                cnt = jnp.max(plsc.all_reduce_population_count(m2))
                return off2 + cnt, zc + cnt
            return lax.fori_loop(0, NVREG_ROW, zbody, (off, 0))[0]

        off2 = lax.cond(nnz < M, zfill, lambda: off)
        nv = (off2 + 15) >> 4

        def leaf1(j, _):
            kk, vv = plsc.sort_key_val(
                valbuf[pl.ds(j * 16, 16)], idxbuf[pl.ds(j * 16, 16)],
                descending=True)
            valbuf[pl.ds(j * 16, 16)] = kk
            idxbuf[pl.ds(j * 16, 16)] = vv
            return 0
        lax.fori_loop(0, nv, leaf1, 0)

        def pass1(p, _):
            par = lax.rem(p, 2)

            def pair(jj, _):
                j = 2 * jj + par

                @pl.when(j + 1 < nv)
                def _():
                    ak = valbuf[pl.ds(j * 16, 16)]
                    av = idxbuf[pl.ds(j * 16, 16)]
                    bk = valbuf[pl.ds(j * 16 + 16, 16)]
                    bv = idxbuf[pl.ds(j * 16 + 16, 16)]
                    rbk = lax.rev(bk, (0,))
                    rbv = lax.rev(bv, (0,))
                    m = ak >= rbk
                    hk = jnp.where(m, ak, rbk)
                    hv = jnp.where(m, av, rbv)
                    lk = jnp.where(m, rbk, ak)
                    lv = jnp.where(m, rbv, av)
                    hk, hv = plsc.sort_key_val(hk, hv, descending=True)
                    lk, lv = plsc.sort_key_val(lk, lv, descending=True)
                    valbuf[pl.ds(j * 16, 16)] = hk
                    idxbuf[pl.ds(j * 16, 16)] = hv
                    valbuf[pl.ds(j * 16 + 16, 16)] = lk
                    idxbuf[pl.ds(j * 16 + 16, 16)] = lv
                return 0
            lax.fori_loop(0, (nv + 1) >> 1, pair, 0)
            return 0
        lax.fori_loop(0, nv, pass1, 0)

        shbuf[pl.ds(0, 16)] = jnp.full((16,), -1.0, jnp.float32)

        def shift_store(j, _):
            shbuf[pl.ds(j * 16 + 1, 16)] = valbuf[pl.ds(j * 16, 16)]
            return 0
        lax.fori_loop(0, nv, shift_store, 0)

        def rid_body(j, rc):
            kk = valbuf[pl.ds(j * 16, 16)]
            pv = shbuf[pl.ds(j * 16, 16)]
            neq = jnp.where(kk != pv, 1, 0)
            cs = plsc.cumsum(neq)
            rid = rc + cs
            plsc.store_scatter(tbl, [rid], kk)
            keybuf[pl.ds(j * 16, 16)] = rid * 16384 + idxbuf[pl.ds(j * 16, 16)]
            return rc + jnp.max(cs)
        lax.fori_loop(0, nv, rid_body, -1)

        def leaf2(j, _):
            kk, _vv = plsc.sort_key_val(
                keybuf[pl.ds(j * 16, 16)], keybuf[pl.ds(j * 16, 16)],
                descending=False)
            keybuf[pl.ds(j * 16, 16)] = kk
            return 0
        lax.fori_loop(0, nv, leaf2, 0)

        def pass2(p, _):
            par = lax.rem(p, 2)

            def pair(jj, _):
                j = 2 * jj + par

                @pl.when(j + 1 < nv)
                def _():
                    ak = keybuf[pl.ds(j * 16, 16)]
                    bk = keybuf[pl.ds(j * 16 + 16, 16)]
                    rbk = lax.rev(bk, (0,))
                    m = ak <= rbk
                    lk = jnp.where(m, ak, rbk)
                    hk = jnp.where(m, rbk, ak)
                    lk, _l = plsc.sort_key_val(lk, lk, descending=False)
                    hk, _h = plsc.sort_key_val(hk, hk, descending=False)
                    keybuf[pl.ds(j * 16, 16)] = lk
                    keybuf[pl.ds(j * 16 + 16, 16)] = hk
                return 0
            lax.fori_loop(0, (nv + 1) >> 1, pair, 0)
            return 0
        lax.fori_loop(0, nv, pass2, 0)

        def decode(j, _):
            key = keybuf[pl.ds(j * 16, 16)]
            idx = key & 16383
            rid = key >> 14
            val = plsc.load_gather(tbl, [rid])
            evec = _iota16() + j * 16
            for tcol in range(D_IN):
                word = (row + half * (B // 2)) * (D_IN * N) + tcol * N + idx
                slot = evec * D_IN + tcol
                plsc.store_scatter(
                    idxw, [slot >> 7, slot & 127], word)
                plsc.store_scatter(
                    valb, [evec, jnp.full((16,), tcol, jnp.int32)], val)
            return 0
        lax.fori_loop(0, M // 16, decode, 0)

        def fire(g, _):
            pltpu.async_copy(inpw_hbm.at[idxw.at[g]],
                             rowsw.at[pl.ds(g * 128, 128)], sem)
            return 0
        lax.fori_loop(0, 16, fire, 0)

        def drain(g, _):
            pltpu.make_async_copy(inpw_hbm.at[idxw.at[g]],
                                  rowsw.at[pl.ds(g * 128, 128)], sem).wait()
            return 0
        lax.fori_loop(0, 16, drain, 0)

        pltpu.sync_copy(rowsw, rows_out.at[pl.ds(row * M * D_IN, M * D_IN)])
        pltpu.sync_copy(valb, valsb_out.at[pl.ds(row * M, M)])
        return 0

    lax.fori_loop(0, 2, do_row, 0)


def _sc_topk(p_flat, inpw, half):
    mesh = plsc.VectorSubcoreMesh(core_axis_name="c", subcore_axis_name="s")
    f32 = jnp.float32
    return pl.kernel(
        functools.partial(_sc_body, half),
        mesh=mesh,
        compiler_params=pltpu.CompilerParams(
            needs_layout_passes=False, use_tc_tiling_on_sc=False),
        out_type=(
            jax.ShapeDtypeStruct((B // 2 * M, D_IN), f32),
            jax.ShapeDtypeStruct((B // 2 * M * D_IN,), f32),
        ),
        scratch_types=[
            pltpu.VMEM((N,), f32),
            pltpu.VMEM((272,), jnp.int32),
            pltpu.VMEM((CAP,), f32),
            pltpu.VMEM((CAP,), jnp.int32),
            pltpu.VMEM((CAP + 16,), f32),
            pltpu.VMEM((CAP,), jnp.int32),
            pltpu.VMEM((CAP,), f32),
            pltpu.VMEM((M * D_IN // 128, 128), jnp.int32),
            pltpu.VMEM((M * D_IN,), f32),
            pltpu.VMEM((M, D_IN), f32),
            pltpu.SemaphoreType.DMA,
        ],
    )(p_flat, inpw)


def _logmul_body(v_ref, r_ref, out_ref):
    la = jnp.log(jnp.minimum(jnp.maximum(v_ref[...], 1e-8), 1.0))
    out_ref[...] = r_ref[...] * la


def _logmul(valsb, rows):
    blk = 2048
    return pl.pallas_call(
        _logmul_body,
        grid=(B * M // blk,),
        in_specs=[
            pl.BlockSpec((blk, D_IN), lambda i: (i, 0)),
            pl.BlockSpec((blk, D_IN), lambda i: (i, 0)),
        ],
        out_specs=pl.BlockSpec((blk, D_IN), lambda i: (i, 0)),
        out_shape=jax.ShapeDtypeStruct((B * M, D_IN), jnp.float32),
    )(valsb, rows)


def kernel(states, inputs, W, b, k):
    shift = (jnp.asarray(k) - M).astype(jnp.float32)
    sT3 = jnp.transpose(states, (2, 0, 1))
    inpw = jnp.transpose(inputs, (0, 2, 1)).reshape(B * D_IN * N)
    P0 = _predicts(sT3, W, b, shift, 0)
    v0, r0 = _sc_topk(P0.reshape(BN // 2), inpw, 0)
    P1 = _predicts(sT3, W, b, shift, 1)
    v1, r1 = _sc_topk(P1.reshape(BN // 2), inpw, 1)
    valsb = jnp.concatenate([v0, v1], axis=0)
    rows = jnp.concatenate([r0, r1], axis=0)
    return _logmul(valsb, rows.reshape(B * M, D_IN)).reshape(B, M, D_IN)


# speedup vs baseline: 3.2398x; 1.0040x over previous
import functools

import jax
import jax.numpy as jnp
from jax import lax
from jax.experimental import pallas as pl
from jax.experimental.pallas import tpu as pltpu
from jax.experimental.pallas import tpu_sc as plsc

B, N, D_STATE, D_IN, M = 128, 8192, 25, 16, 128
BN = B * N
NB = 4096

CAP = 512
NVREG_ROW = N // 16


def _pred_body(w_ref, b_ref, shift_ref, x_ref, out_ref):
    x = x_ref[...]
    xb = x.astype(jnp.bfloat16)
    wb = w_ref[...].astype(jnp.bfloat16)
    outs = []
    for s in range(8):
        rhs = xb[:, s, :]
        outs.append(lax.dot_general(
            wb, rhs,
            dimension_numbers=(((1,), (0,)), ((), ())),
            preferred_element_type=jnp.float32,
        ))
    logits = jnp.concatenate(outs, axis=0) + b_ref[0, 0]
    p = 1.0 / (1.0 + jnp.exp(-logits))
    out_ref[...] = jnp.where(p >= 0.5, p, 0.0) + shift_ref[0, 0]


def _predicts(sT3, W, b, shift, half):
    hb = half * (B // 2 // 8)
    return pl.pallas_call(
        _pred_body,
        grid=(B // 2 // 8, N // NB),
        in_specs=[
            pl.BlockSpec((1, D_STATE), lambda i, j: (0, 0)),
            pl.BlockSpec((1, 1), lambda i, j: (0, 0)),
            pl.BlockSpec((1, 1), lambda i, j: (0, 0)),
            pl.BlockSpec((D_STATE, 8, NB), lambda i, j, hb=hb: (0, hb + i, j)),
        ],
        out_specs=pl.BlockSpec((8, NB), lambda i, j: (i, j)),
        out_shape=jax.ShapeDtypeStruct((B // 2, N), jnp.float32),
    )(W.reshape(1, D_STATE), b.reshape(1, 1), shift.reshape(1, 1), sT3)


def _iota16():
    return lax.iota(jnp.int32, 16)


def _sc_body(half, p_hbm, inpw_hbm, valsb_out, rows_out,
             prow, hist, valbuf, idxbuf, shbuf, keybuf, tbl,
             idxw, rowsw, valb, sem):
    nc = 2
    wid = lax.axis_index("s") * nc + lax.axis_index("c")

    def do_row(t, _):
        row = wid * 2 + t
        pltpu.sync_copy(p_hbm.at[pl.ds(row * N, N)], prow)

        def zero_hist(j, _):
            hist[pl.ds(j * 16, 16)] = jnp.zeros((16,), jnp.int32)
            return 0
        lax.fori_loop(0, 17, zero_hist, 0)

        def init_buf(j, _):
            valbuf[pl.ds(j * 16, 16)] = jnp.zeros((16,), jnp.float32)
            idxbuf[pl.ds(j * 16, 16)] = jnp.full((16,), 12288, jnp.int32)
            return 0
        lax.fori_loop(0, CAP // 16, init_buf, 0)

        @plsc.parallel_loop(0, NVREG_ROW, unroll=16)
        def hist_body(i):
            v = prow[pl.ds(i * 16, 16)]
            bits = lax.bitcast_convert_type(v, jnp.int32)
            key = jnp.maximum((bits >> 15) - 32255, 0)
            plsc.addupdate_scatter(hist, [key], jnp.ones((16,), jnp.int32))

        def scan_body(jj, carry):
            carry_sum, tbin = carry
            j = 16 - jj
            h = hist[pl.ds(j * 16, 16)]
            binid = _iota16() + j * 16
            cs = plsc.cumsum(h)
            tot = jnp.max(cs)
            suffix = carry_sum + tot - cs + h
            cand = jnp.where(suffix >= M, binid, -1)
            return carry_sum + tot, jnp.maximum(tbin, jnp.max(cand))
        _, tbin = lax.fori_loop(0, 17, scan_body, (0, -1))
        t1 = jnp.maximum(tbin, 1)

        def cnt_body(j, carry):
            h = hist[pl.ds(j * 16, 16)]
            binid = _iota16() + j * 16
            return carry + jnp.sum(jnp.where(binid >= 1, h, 0))
        nnz = lax.fori_loop(0, 17, cnt_body, 0)

        def compact_body(i, off):
            v = prow[pl.ds(i * 16, 16)]
            bits = lax.bitcast_convert_type(v, jnp.int32)
            key = jnp.minimum(jnp.maximum((bits >> 15) - 32255, 0), 257)
            m = (key >= t1) & (off < CAP - 16)
            plsc.store_compressed(valbuf.at[pl.ds(off, 16)], v, mask=m)
            ivec = _iota16() + i * 16
            plsc.store_compressed(idxbuf.at[pl.ds(off, 16)], ivec, mask=m)
            cnt = jnp.max(plsc.all_reduce_population_count(m))
            return off + cnt
        off = lax.fori_loop(0, NVREG_ROW, compact_body, 0, unroll=8)

        def zfill():
            need = M - nnz

            def zbody(i, carry):
                off2, zc = carry
                v = prow[pl.ds(i * 16, 16)]
                bits = lax.bitcast_convert_type(v, jnp.int32)
                mz = (bits == 0) & (off2 < CAP - 16)
                rank = plsc.cumsum(jnp.where(mz, 1, 0))
                m2 = mz & (zc + rank <= need)
                plsc.store_compressed(valbuf.at[pl.ds(off2, 16)], v, mask=m2)
                ivec = _iota16() + i * 16
                plsc.store_compressed(idxbuf.at[pl.ds(off2, 16)], ivec, mask=m2)
                cnt = jnp.max(plsc.all_reduce_population_count(m2))
                return off2 + cnt, zc + cnt
            return lax.fori_loop(0, NVREG_ROW, zbody, (off, 0))[0]

        off2 = lax.cond(nnz < M, zfill, lambda: off)
        nv = (off2 + 15) >> 4

        def leaf1(j, _):
            kk, vv = plsc.sort_key_val(
                valbuf[pl.ds(j * 16, 16)], idxbuf[pl.ds(j * 16, 16)],
                descending=True)
            valbuf[pl.ds(j * 16, 16)] = kk
            idxbuf[pl.ds(j * 16, 16)] = vv
            return 0
        lax.fori_loop(0, nv, leaf1, 0)

        def pass1(p, _):
            par = lax.rem(p, 2)

            def pair(jj, _):
                j = 2 * jj + par

                @pl.when(j + 1 < nv)
                def _():
                    ak = valbuf[pl.ds(j * 16, 16)]
                    av = idxbuf[pl.ds(j * 16, 16)]
                    bk = valbuf[pl.ds(j * 16 + 16, 16)]
                    bv = idxbuf[pl.ds(j * 16 + 16, 16)]
                    rbk = lax.rev(bk, (0,))
                    rbv = lax.rev(bv, (0,))
                    m = ak >= rbk
                    hk = jnp.where(m, ak, rbk)
                    hv = jnp.where(m, av, rbv)
                    lk = jnp.where(m, rbk, ak)
                    lv = jnp.where(m, rbv, av)
                    hk, hv = plsc.sort_key_val(hk, hv, descending=True)
                    lk, lv = plsc.sort_key_val(lk, lv, descending=True)
                    valbuf[pl.ds(j * 16, 16)] = hk
                    idxbuf[pl.ds(j * 16, 16)] = hv
                    valbuf[pl.ds(j * 16 + 16, 16)] = lk
                    idxbuf[pl.ds(j * 16 + 16, 16)] = lv
                return 0
            lax.fori_loop(0, (nv + 1) >> 1, pair, 0)
            return 0
        lax.fori_loop(0, nv, pass1, 0)

        shbuf[pl.ds(0, 16)] = jnp.full((16,), -1.0, jnp.float32)

        def shift_store(j, _):
            shbuf[pl.ds(j * 16 + 1, 16)] = valbuf[pl.ds(j * 16, 16)]
            return 0
        lax.fori_loop(0, nv, shift_store, 0)

        def rid_body(j, rc):
            kk = valbuf[pl.ds(j * 16, 16)]
            pv = shbuf[pl.ds(j * 16, 16)]
            neq = jnp.where(kk != pv, 1, 0)
            cs = plsc.cumsum(neq)
            rid = rc + cs
            plsc.store_scatter(tbl, [rid], kk)
            keybuf[pl.ds(j * 16, 16)] = rid * 16384 + idxbuf[pl.ds(j * 16, 16)]
            return rc + jnp.max(cs)
        lax.fori_loop(0, nv, rid_body, -1)

        def leaf2(j, _):
            kk, _vv = plsc.sort_key_val(
                keybuf[pl.ds(j * 16, 16)], keybuf[pl.ds(j * 16, 16)],
                descending=False)
            keybuf[pl.ds(j * 16, 16)] = kk
            return 0
        lax.fori_loop(0, nv, leaf2, 0)

        def pass2(p, _):
            par = lax.rem(p, 2)

            def pair(jj, _):
                j = 2 * jj + par

                @pl.when(j + 1 < nv)
                def _():
                    ak = keybuf[pl.ds(j * 16, 16)]
                    bk = keybuf[pl.ds(j * 16 + 16, 16)]
                    rbk = lax.rev(bk, (0,))
                    m = ak <= rbk
                    lk = jnp.where(m, ak, rbk)
                    hk = jnp.where(m, rbk, ak)
                    lk, _l = plsc.sort_key_val(lk, lk, descending=False)
                    hk, _h = plsc.sort_key_val(hk, hk, descending=False)
                    keybuf[pl.ds(j * 16, 16)] = lk
                    keybuf[pl.ds(j * 16 + 16, 16)] = hk
                return 0
            lax.fori_loop(0, (nv + 1) >> 1, pair, 0)
            return 0
        lax.fori_loop(0, nv, pass2, 0)

        def decode(j, _):
            key = keybuf[pl.ds(j * 16, 16)]
            idx = key & 16383
            rid = key >> 14
            val = plsc.load_gather(tbl, [rid])
            evec = _iota16() + j * 16
            for tcol in range(D_IN):
                word = (row + half * (B // 2)) * (D_IN * N) + tcol * N + idx
                slot = evec * D_IN + tcol
                plsc.store_scatter(
                    idxw, [slot >> 7, slot & 127], word)
                plsc.store_scatter(
                    valb, [evec, jnp.full((16,), tcol, jnp.int32)], val)
            return 0
        lax.fori_loop(0, M // 16, decode, 0)

        def fire(g, _):
            pltpu.async_copy(inpw_hbm.at[idxw.at[g]],
                             rowsw.at[pl.ds(g * 128, 128)], sem)
            return 0
        lax.fori_loop(0, 16, fire, 0)

        def drain(g, _):
            pltpu.make_async_copy(inpw_hbm.at[idxw.at[g]],
                                  rowsw.at[pl.ds(g * 128, 128)], sem).wait()
            return 0
        lax.fori_loop(0, 16, drain, 0)

        pltpu.sync_copy(rowsw, rows_out.at[pl.ds(row * M * D_IN, M * D_IN)])
        pltpu.sync_copy(valb, valsb_out.at[pl.ds(row * M, M)])
        return 0

    lax.fori_loop(0, 2, do_row, 0)


def _sc_topk(p_flat, inpw, half):
    mesh = plsc.VectorSubcoreMesh(core_axis_name="c", subcore_axis_name="s")
    f32 = jnp.float32
    return pl.kernel(
        functools.partial(_sc_body, half),
        mesh=mesh,
        compiler_params=pltpu.CompilerParams(
            needs_layout_passes=False, use_tc_tiling_on_sc=False),
        out_type=(
            jax.ShapeDtypeStruct((B // 2 * M, D_IN), f32),
            jax.ShapeDtypeStruct((B // 2 * M * D_IN,), f32),
        ),
        scratch_types=[
            pltpu.VMEM((N,), f32),
            pltpu.VMEM((272,), jnp.int32),
            pltpu.VMEM((CAP,), f32),
            pltpu.VMEM((CAP,), jnp.int32),
            pltpu.VMEM((CAP + 16,), f32),
            pltpu.VMEM((CAP,), jnp.int32),
            pltpu.VMEM((CAP,), f32),
            pltpu.VMEM((M * D_IN // 128, 128), jnp.int32),
            pltpu.VMEM((M * D_IN,), f32),
            pltpu.VMEM((M, D_IN), f32),
            pltpu.SemaphoreType.DMA,
        ],
    )(p_flat, inpw)


def _logmul_body(v_ref, r_ref, out_ref):
    la = jnp.log(jnp.minimum(jnp.maximum(v_ref[...], 1e-8), 1.0))
    out_ref[...] = r_ref[...] * la


def _logmul(valsb, rows):
    blk = 2048
    return pl.pallas_call(
        _logmul_body,
        grid=(B * M // blk,),
        in_specs=[
            pl.BlockSpec((blk, D_IN), lambda i: (i, 0)),
            pl.BlockSpec((blk, D_IN), lambda i: (i, 0)),
        ],
        out_specs=pl.BlockSpec((blk, D_IN), lambda i: (i, 0)),
        out_shape=jax.ShapeDtypeStruct((B * M, D_IN), jnp.float32),
    )(valsb, rows)


def kernel(states, inputs, W, b, k):
    shift = (jnp.asarray(k) - M).astype(jnp.float32)
    sT3 = jnp.transpose(states, (2, 0, 1))
    inpw = jnp.transpose(inputs, (0, 2, 1)).reshape(B * D_IN * N)
    P0 = _predicts(sT3, W, b, shift, 0)
    v0, r0 = _sc_topk(P0.reshape(BN // 2), inpw, 0)
    P1 = _predicts(sT3, W, b, shift, 1)
    v1, r1 = _sc_topk(P1.reshape(BN // 2), inpw, 1)
    valsb = jnp.concatenate([v0, v1], axis=0)
    rows = jnp.concatenate([r0, r1], axis=0)
    return _logmul(valsb, rows.reshape(B * M, D_IN)).reshape(B, M, D_IN)
